# baseline (speedup 1.0000x reference)
import functools

import jax
import jax.numpy as jnp
from jax import lax
from jax.experimental import pallas as pl
from jax.experimental.pallas import tpu as pltpu
from jax.experimental.pallas import tpu_sc as plsc

L = 4096
D = 64
NUM_BLOCKS = 16
BLOCK_DIM = 4
V = 512
NTILES = 32
RPT = L // NTILES
TW = D + 16


def _norm_body(x_ref, tbn_ref, q_ref):
    x = x_ref[...]
    x2 = x * x
    lane = lax.broadcasted_iota(jnp.int32, (D, NUM_BLOCKS), 0)
    blk = lax.broadcasted_iota(jnp.int32, (D, NUM_BLOCKS), 1)
    g = (lane // BLOCK_DIM == blk).astype(jnp.float32)
    ss = lax.dot_general(x2, g, (((1,), (0,)), ((), ())),
                         precision=lax.Precision.HIGHEST)
    nrm = jnp.maximum(jnp.sqrt(ss), 1e-12)
    nexp = lax.dot_general(nrm, g.T, (((1,), (0,)), ((), ())),
                           precision=lax.Precision.HIGHEST)
    tbn = x / nexp
    tbn_ref[...] = tbn
    q_ref[...] = jnp.sum(tbn * tbn).reshape(1, 1)


def _finish_body(p_ref, q_ref, out_ref):
    p = jnp.sum(p_ref[...], axis=0)
    s = p[:, :D]
    c = p[:, D:]
    rowsq = jnp.sum(s * s, axis=1, keepdims=True)
    diff = jnp.sum(rowsq - NUM_BLOCKS * c[:, :1])
    pairs = jnp.sum(c * c - c) / (2.0 * 16.0)
    q = jnp.sum(q_ref[...])
    total = (diff + (NUM_BLOCKS * L - q)) / (2.0 * NUM_BLOCKS)
    out_ref[...] = jnp.where(pairs > 0.5, total / pairs, 0.0).reshape(1, 1)


def _sc_scatter_body(tbn_hbm, tok_hbm, out_tab, rows_v, idx_v, table_v):
    cid = lax.axis_index("c")
    sid = lax.axis_index("s")
    wid = cid * 16 + sid
    base = wid * RPT
    pltpu.sync_copy(tok_hbm.at[pl.ds(base, RPT)], idx_v)
    pltpu.sync_copy(tbn_hbm.at[pl.ds(base, RPT)], rows_v)

    z = jnp.zeros((16,), jnp.float32)

    def zbody(i, _):
        for k in range(TW // 16):
            table_v[i, pl.ds(k * 16, 16)] = z
        return 0

    lax.fori_loop(0, V, zbody, 0)

    ones = jnp.ones((16,), jnp.float32)

    def body(grp, _):
        tv = idx_v[pl.ds(grp * 16, 16)]
        for j in range(16):
            t = tv[j]
            r = grp * 16 + j
            for k in range(D // 16):
                v = rows_v[r, pl.ds(k * 16, 16)]
                plsc.addupdate(table_v.at[t, pl.ds(k * 16, 16)], v)
            plsc.addupdate(table_v.at[t, pl.ds(D, 16)], ones)
        return 0

    lax.fori_loop(0, RPT // 16, body, 0)
    pltpu.sync_copy(table_v, out_tab.at[wid])


_sc_scatter = functools.partial(
    pl.kernel,
    out_type=jax.ShapeDtypeStruct((NTILES, V, TW), jnp.float32),
    mesh=plsc.VectorSubcoreMesh(core_axis_name="c", subcore_axis_name="s"),
    scratch_types=[
        pltpu.VMEM((RPT, D), jnp.float32),
        pltpu.VMEM((RPT,), jnp.int32),
        pltpu.VMEM((V, TW), jnp.float32),
    ],
)(_sc_scatter_body)


def kernel(semantic_state, token_ids):
    x = semantic_state.reshape(L, D)
    tok = token_ids.reshape(L)

    tbn, q = pl.pallas_call(
        _norm_body,
        out_shape=[
            jax.ShapeDtypeStruct((L, D), jnp.float32),
            jax.ShapeDtypeStruct((1, 1), jnp.float32),
        ],
    )(x)

    tables = _sc_scatter(tbn, tok)

    loss = pl.pallas_call(
        _finish_body,
        out_shape=jax.ShapeDtypeStruct((1, 1), jnp.float32),
    )(tables, q)
    return loss.reshape(())
